# baseline (speedup 1.0000x reference)
import functools

import jax
import jax.numpy as jnp
from jax import lax
from jax.experimental import pallas as pl
from jax.experimental.pallas import tpu as pltpu
from jax.experimental.pallas import tpu_sc as plsc

N = 10000
NP = 10240
D = 128
E = 320000

NC = 2
NS = 16
NW = NC * NS
L = 16

K = 128
NCHUNK = 2624
Q0 = 148
Q1 = NCHUNK // NS - Q0
NITM = max(Q0, Q1)
EPAD = NCHUNK * K
RPT = NP // NS

_mesh = plsc.VectorSubcoreMesh(core_axis_name="c", subcore_axis_name="s")


def _zero_acc(gath, acc, s):
    def zrow(r, carry):
        zv = lax.broadcast(jnp.float32(0.0), (L,))
        for j in range(D // L):
            gath[r, pl.ds(j * L, L)] = zv
        return carry

    lax.fori_loop(0, K, zrow, 0)
    for t in range(RPT // K):
        pltpu.sync_copy(gath, acc.at[pl.ds(s * RPT + t * K, K)])


def _scale(gath, pk):
    def grp(g, carry):
        evec = pk[2, pl.ds(g * L, L)]
        for t in range(L):
            wv = lax.broadcast(
                lax.bitcast_convert_type(evec[t], jnp.float32), (L,))
            e = g * L + t
            for j in range(D // L):
                gath[e, pl.ds(j * L, L)] = gath[e, pl.ds(j * L, L)] * wv
        return carry

    lax.fori_loop(0, K // L, grp, 0)


def _splat(gath, pk):
    def grp(g, carry):
        evec = pk[2, pl.ds(g * L, L)]
        for t in range(L):
            wv = lax.broadcast(
                lax.bitcast_convert_type(evec[t], jnp.float32), (L,))
            e = g * L + t
            for j in range(D // L):
                gath[e, pl.ds(j * L, L)] = wv
        return carry

    lax.fori_loop(0, K // L, grp, 0)


def _agg_body(z, pk3, out, pk0, pk1, g0, g1, acc, gs0, gs1, ss0, ss1):
    c = lax.axis_index("c")
    s = lax.axis_index("s")
    w = s * NC + c
    npair = jnp.where(c == 0, Q0 // 2 - 1, Q1 // 2 - 1)
    _zero_acc(g0, acc, s)
    plsc.subcore_barrier()

    pltpu.sync_copy(pk3.at[w, 0], pk0)
    pltpu.async_copy(z.at[pk0.at[0]], g0, gs0)
    pltpu.sync_copy(pk3.at[w, 1], pk1)
    pltpu.async_copy(z.at[pk1.at[0]], g1, gs1)

    def pair(j, carry):
        i0 = 2 * j
        pltpu.make_async_copy(z.at[pk0.at[0]], g0, gs0).wait()
        _scale(g0, pk0)
        pltpu.async_copy(g0, acc.at[pk0.at[1]], ss0, add=True)
        pltpu.make_async_copy(z.at[pk1.at[0]], g1, gs1).wait()
        _scale(g1, pk1)
        pltpu.async_copy(g1, acc.at[pk1.at[1]], ss1, add=True)
        pltpu.make_async_copy(g0, acc.at[pk0.at[1]], ss0).wait()
        pltpu.sync_copy(pk3.at[w, i0 + 2], pk0)
        pltpu.async_copy(z.at[pk0.at[0]], g0, gs0)
        pltpu.make_async_copy(g1, acc.at[pk1.at[1]], ss1).wait()
        pltpu.sync_copy(pk3.at[w, i0 + 3], pk1)
        pltpu.async_copy(z.at[pk1.at[0]], g1, gs1)
        return carry

    lax.fori_loop(0, npair, pair, 0)

    pltpu.make_async_copy(z.at[pk0.at[0]], g0, gs0).wait()
    _scale(g0, pk0)
    pltpu.async_copy(g0, acc.at[pk0.at[1]], ss0, add=True)
    pltpu.make_async_copy(z.at[pk1.at[0]], g1, gs1).wait()
    _scale(g1, pk1)
    pltpu.async_copy(g1, acc.at[pk1.at[1]], ss1, add=True)
    pltpu.make_async_copy(g0, acc.at[pk0.at[1]], ss0).wait()
    pltpu.make_async_copy(g1, acc.at[pk1.at[1]], ss1).wait()

    plsc.subcore_barrier()
    pltpu.sync_copy(acc.at[pl.ds(s * RPT, RPT)],
                    out.at[c, pl.ds(s * RPT, RPT)])


_agg_call = pl.kernel(
    _agg_body,
    out_type=jax.ShapeDtypeStruct((NC, NP, D), jnp.float32),
    mesh=_mesh,
    scratch_types=[
        pltpu.VMEM((3, K), jnp.int32),
        pltpu.VMEM((3, K), jnp.int32),
        pltpu.VMEM((K, D), jnp.float32),
        pltpu.VMEM((K, D), jnp.float32),
        pltpu.VMEM_SHARED((NP, D), jnp.float32),
        pltpu.SemaphoreType.DMA,
        pltpu.SemaphoreType.DMA,
        pltpu.SemaphoreType.DMA,
        pltpu.SemaphoreType.DMA,
    ],
)


def _deg_body(pk3, out, pk0, pk1, g0, g1, acc, ss0, ss1):
    c = lax.axis_index("c")
    s = lax.axis_index("s")
    w = s * NC + c
    npair = jnp.where(c == 0, Q0 // 2 - 1, Q1 // 2 - 1)
    _zero_acc(g0, acc, s)
    plsc.subcore_barrier()

    pltpu.sync_copy(pk3.at[w, 0], pk0)
    _splat(g0, pk0)
    pltpu.async_copy(g0, acc.at[pk0.at[1]], ss0, add=True)
    pltpu.sync_copy(pk3.at[w, 1], pk1)
    _splat(g1, pk1)
    pltpu.async_copy(g1, acc.at[pk1.at[1]], ss1, add=True)

    def pair(j, carry):
        i0 = 2 * j
        pltpu.make_async_copy(g0, acc.at[pk0.at[1]], ss0).wait()
        pltpu.sync_copy(pk3.at[w, i0 + 2], pk0)
        _splat(g0, pk0)
        pltpu.async_copy(g0, acc.at[pk0.at[1]], ss0, add=True)
        pltpu.make_async_copy(g1, acc.at[pk1.at[1]], ss1).wait()
        pltpu.sync_copy(pk3.at[w, i0 + 3], pk1)
        _splat(g1, pk1)
        pltpu.async_copy(g1, acc.at[pk1.at[1]], ss1, add=True)
        return carry

    lax.fori_loop(0, npair, pair, 0)
    pltpu.make_async_copy(g0, acc.at[pk0.at[1]], ss0).wait()
    pltpu.make_async_copy(g1, acc.at[pk1.at[1]], ss1).wait()

    plsc.subcore_barrier()
    pltpu.sync_copy(acc.at[pl.ds(s * RPT, RPT)],
                    out.at[c, pl.ds(s * RPT, RPT)])


_deg_call = pl.kernel(
    _deg_body,
    out_type=jax.ShapeDtypeStruct((NC, NP, D), jnp.float32),
    mesh=_mesh,
    scratch_types=[
        pltpu.VMEM((3, K), jnp.int32),
        pltpu.VMEM((3, K), jnp.int32),
        pltpu.VMEM((K, D), jnp.float32),
        pltpu.VMEM((K, D), jnp.float32),
        pltpu.VMEM_SHARED((NP, D), jnp.float32),
        pltpu.SemaphoreType.DMA,
        pltpu.SemaphoreType.DMA,
    ],
)


B = 1024


def _dis(degp_blk):
    d = degp_blk[0, :, 0:1] + degp_blk[1, :, 0:1]
    return jnp.where(d > 0, lax.rsqrt(d), 0.0)


def _mm_first_body(x_ref, w_ref, degp_ref, z_ref):
    dis = _dis(degp_ref[...])
    z_ref[...] = jnp.dot(x_ref[...], w_ref[...],
                         preferred_element_type=jnp.float32) * dis


def _mm_mid_body(p_ref, degp_ref, b_ref, w_ref, z_ref):
    dis = _dis(degp_ref[...])
    h = jnp.maximum(dis * (p_ref[0] + p_ref[1]) + b_ref[...], 0.0)
    z_ref[...] = jnp.dot(h, w_ref[...],
                         preferred_element_type=jnp.float32) * dis


def _finish_body(p_ref, degp_ref, b_ref, o_ref):
    dis = _dis(degp_ref[...])
    o_ref[...] = dis * (p_ref[0] + p_ref[1]) + b_ref[...]


_grid = (NP // B,)
_bs_x = pl.BlockSpec((B, D), lambda i: (i, 0))
_bs_w = pl.BlockSpec((D, D), lambda i: (0, 0))
_bs_b = pl.BlockSpec((1, D), lambda i: (0, 0))
_bs_degp = pl.BlockSpec((2, B, D), lambda i: (0, i, 0))
_bs_p = pl.BlockSpec((2, B, D), lambda i: (0, i, 0))

_mm_first = pl.pallas_call(
    _mm_first_body, grid=_grid,
    in_specs=[_bs_x, _bs_w, _bs_degp], out_specs=_bs_x,
    out_shape=jax.ShapeDtypeStruct((NP, D), jnp.float32))

_mm_mid = pl.pallas_call(
    _mm_mid_body, grid=_grid,
    in_specs=[_bs_p, _bs_degp, _bs_b, _bs_w], out_specs=_bs_x,
    out_shape=jax.ShapeDtypeStruct((NP, D), jnp.float32))

_finish = pl.pallas_call(
    _finish_body, grid=_grid,
    in_specs=[_bs_p, _bs_degp, _bs_b], out_specs=_bs_x,
    out_shape=jax.ShapeDtypeStruct((NP, D), jnp.float32))


def kernel(x, edge_index, edge_weight, W1, b1, W2, b2, W3, b3, W4, b4):
    loop = jnp.arange(N, dtype=edge_index.dtype)
    row = jnp.concatenate([edge_index[0], loop])
    col = jnp.concatenate([edge_index[1], loop])
    ew = jnp.concatenate([edge_weight, jnp.ones((N,), jnp.float32)])
    pad = EPAD - (E + N)
    rowc = jnp.pad(row, (0, pad)).reshape(NCHUNK, K)
    colc = jnp.pad(col, (0, pad)).reshape(NCHUNK, K)
    ewbc = lax.bitcast_convert_type(
        jnp.pad(ew, (0, pad)), jnp.int32).reshape(NCHUNK, K)
    pkc = jnp.stack([rowc, colc, ewbc], axis=1)
    n0 = NS * Q0
    pk_a = jnp.pad(pkc[:n0].reshape(NS, Q0, 3, K),
                   ((0, 0), (0, NITM - Q0), (0, 0), (0, 0)))
    pk_b = jnp.pad(pkc[n0:].reshape(NS, Q1, 3, K),
                   ((0, 0), (0, NITM - Q1), (0, 0), (0, 0)))
    pk3 = jnp.stack([pk_a, pk_b], axis=1).reshape(NW, NITM, 3, K)
    xp = jnp.pad(x, ((0, NP - N), (0, 0)))

    degp = _deg_call(pk3)

    b1r, b2r, b3r, b4r = (b.reshape(1, D) for b in (b1, b2, b3, b4))
    z = _mm_first(xp, W1, degp)
    p = _agg_call(z, pk3)
    z = _mm_mid(p, degp, b1r, W2)
    p = _agg_call(z, pk3)
    z = _mm_mid(p, degp, b2r, W3)
    p = _agg_call(z, pk3)
    z = _mm_mid(p, degp, b3r, W4)
    p = _agg_call(z, pk3)
    return _finish(p, degp, b4r)[:N]


# speedup vs baseline: 1.0271x; 1.0271x over previous
import functools

import jax
import jax.numpy as jnp
from jax import lax
from jax.experimental import pallas as pl
from jax.experimental.pallas import tpu as pltpu
from jax.experimental.pallas import tpu_sc as plsc

N = 10000
NP = 10240
D = 128
E = 320000

NC = 2
NS = 16
NW = NC * NS
L = 16

K = 128
NCHUNK = 2624
Q0 = 132
Q1 = NCHUNK // NS - Q0
NITM = max(Q0, Q1)
EPAD = NCHUNK * K
RPT = NP // NS

_mesh = plsc.VectorSubcoreMesh(core_axis_name="c", subcore_axis_name="s")


def _zero_acc(gath, acc, s):
    def zrow(r, carry):
        zv = lax.broadcast(jnp.float32(0.0), (L,))
        for j in range(D // L):
            gath[r, pl.ds(j * L, L)] = zv
        return carry

    lax.fori_loop(0, K, zrow, 0)
    for t in range(RPT // K):
        pltpu.sync_copy(gath, acc.at[pl.ds(s * RPT + t * K, K)])


def _scale(gath, pk):
    def grp(g, carry):
        evec = pk[2, pl.ds(g * L, L)]
        for t in range(L):
            wv = lax.broadcast(
                lax.bitcast_convert_type(evec[t], jnp.float32), (L,))
            e = g * L + t
            for j in range(D // L):
                gath[e, pl.ds(j * L, L)] = gath[e, pl.ds(j * L, L)] * wv
        return carry

    lax.fori_loop(0, K // L, grp, 0)


def _splat(gath, pk):
    def grp(g, carry):
        evec = pk[2, pl.ds(g * L, L)]
        for t in range(L):
            wv = lax.broadcast(
                lax.bitcast_convert_type(evec[t], jnp.float32), (L,))
            e = g * L + t
            for j in range(D // L):
                gath[e, pl.ds(j * L, L)] = wv
        return carry

    lax.fori_loop(0, K // L, grp, 0)


def _agg_body(z, pk3, out, pk0, pk1, g0, g1, acc, gs0, gs1, ss0, ss1):
    c = lax.axis_index("c")
    s = lax.axis_index("s")
    w = s * NC + c
    npair = jnp.where(c == 0, Q0 // 2 - 1, Q1 // 2 - 1)
    _zero_acc(g0, acc, s)
    plsc.subcore_barrier()

    pltpu.sync_copy(pk3.at[w, 0], pk0)
    pltpu.async_copy(z.at[pk0.at[0]], g0, gs0)
    pltpu.sync_copy(pk3.at[w, 1], pk1)
    pltpu.async_copy(z.at[pk1.at[0]], g1, gs1)

    def pair(j, carry):
        i0 = 2 * j
        pltpu.make_async_copy(z.at[pk0.at[0]], g0, gs0).wait()
        _scale(g0, pk0)
        pltpu.async_copy(g0, acc.at[pk0.at[1]], ss0, add=True)
        pltpu.make_async_copy(z.at[pk1.at[0]], g1, gs1).wait()
        _scale(g1, pk1)
        pltpu.async_copy(g1, acc.at[pk1.at[1]], ss1, add=True)
        pltpu.make_async_copy(g0, acc.at[pk0.at[1]], ss0).wait()
        pltpu.sync_copy(pk3.at[w, i0 + 2], pk0)
        pltpu.async_copy(z.at[pk0.at[0]], g0, gs0)
        pltpu.make_async_copy(g1, acc.at[pk1.at[1]], ss1).wait()
        pltpu.sync_copy(pk3.at[w, i0 + 3], pk1)
        pltpu.async_copy(z.at[pk1.at[0]], g1, gs1)
        return carry

    lax.fori_loop(0, npair, pair, 0)

    pltpu.make_async_copy(z.at[pk0.at[0]], g0, gs0).wait()
    _scale(g0, pk0)
    pltpu.async_copy(g0, acc.at[pk0.at[1]], ss0, add=True)
    pltpu.make_async_copy(z.at[pk1.at[0]], g1, gs1).wait()
    _scale(g1, pk1)
    pltpu.async_copy(g1, acc.at[pk1.at[1]], ss1, add=True)
    pltpu.make_async_copy(g0, acc.at[pk0.at[1]], ss0).wait()
    pltpu.make_async_copy(g1, acc.at[pk1.at[1]], ss1).wait()

    plsc.subcore_barrier()
    pltpu.sync_copy(acc.at[pl.ds(s * RPT, RPT)],
                    out.at[c, pl.ds(s * RPT, RPT)])


_agg_call = pl.kernel(
    _agg_body,
    out_type=jax.ShapeDtypeStruct((NC, NP, D), jnp.float32),
    mesh=_mesh,
    scratch_types=[
        pltpu.VMEM((3, K), jnp.int32),
        pltpu.VMEM((3, K), jnp.int32),
        pltpu.VMEM((K, D), jnp.float32),
        pltpu.VMEM((K, D), jnp.float32),
        pltpu.VMEM_SHARED((NP, D), jnp.float32),
        pltpu.SemaphoreType.DMA,
        pltpu.SemaphoreType.DMA,
        pltpu.SemaphoreType.DMA,
        pltpu.SemaphoreType.DMA,
    ],
)


QB = NCHUNK // NW


def _deg_body(pk3, out, pk0, pk1, g0, g1, acc, ss0, ss1):
    c = lax.axis_index("c")
    s = lax.axis_index("s")
    w = s * NC + c
    npair = QB // 2 - 1
    _zero_acc(g0, acc, s)
    plsc.subcore_barrier()

    pltpu.sync_copy(pk3.at[w, 0], pk0)
    _splat(g0, pk0)
    pltpu.async_copy(g0, acc.at[pk0.at[1]], ss0, add=True)
    pltpu.sync_copy(pk3.at[w, 1], pk1)
    _splat(g1, pk1)
    pltpu.async_copy(g1, acc.at[pk1.at[1]], ss1, add=True)

    def pair(j, carry):
        i0 = 2 * j
        pltpu.make_async_copy(g0, acc.at[pk0.at[1]], ss0).wait()
        pltpu.sync_copy(pk3.at[w, i0 + 2], pk0)
        _splat(g0, pk0)
        pltpu.async_copy(g0, acc.at[pk0.at[1]], ss0, add=True)
        pltpu.make_async_copy(g1, acc.at[pk1.at[1]], ss1).wait()
        pltpu.sync_copy(pk3.at[w, i0 + 3], pk1)
        _splat(g1, pk1)
        pltpu.async_copy(g1, acc.at[pk1.at[1]], ss1, add=True)
        return carry

    lax.fori_loop(0, npair, pair, 0)
    pltpu.make_async_copy(g0, acc.at[pk0.at[1]], ss0).wait()
    pltpu.make_async_copy(g1, acc.at[pk1.at[1]], ss1).wait()

    plsc.subcore_barrier()
    pltpu.sync_copy(acc.at[pl.ds(s * RPT, RPT)],
                    out.at[c, pl.ds(s * RPT, RPT)])


_deg_call = pl.kernel(
    _deg_body,
    out_type=jax.ShapeDtypeStruct((NC, NP, D), jnp.float32),
    mesh=_mesh,
    scratch_types=[
        pltpu.VMEM((3, K), jnp.int32),
        pltpu.VMEM((3, K), jnp.int32),
        pltpu.VMEM((K, D), jnp.float32),
        pltpu.VMEM((K, D), jnp.float32),
        pltpu.VMEM_SHARED((NP, D), jnp.float32),
        pltpu.SemaphoreType.DMA,
        pltpu.SemaphoreType.DMA,
    ],
)


B = 1024


def _dis(degp_blk):
    d = degp_blk[0, :, 0:1] + degp_blk[1, :, 0:1]
    return jnp.where(d > 0, lax.rsqrt(d), 0.0)


def _mm_first_body(x_ref, w_ref, degp_ref, z_ref):
    dis = _dis(degp_ref[...])
    z_ref[...] = jnp.dot(x_ref[...], w_ref[...],
                         preferred_element_type=jnp.float32) * dis


def _mm_mid_body(p_ref, degp_ref, b_ref, w_ref, z_ref):
    dis = _dis(degp_ref[...])
    h = jnp.maximum(dis * (p_ref[0] + p_ref[1]) + b_ref[...], 0.0)
    z_ref[...] = jnp.dot(h, w_ref[...],
                         preferred_element_type=jnp.float32) * dis


def _finish_body(p_ref, degp_ref, b_ref, o_ref):
    dis = _dis(degp_ref[...])
    o_ref[...] = dis * (p_ref[0] + p_ref[1]) + b_ref[...]


_grid = (NP // B,)
_bs_x = pl.BlockSpec((B, D), lambda i: (i, 0))
_bs_w = pl.BlockSpec((D, D), lambda i: (0, 0))
_bs_b = pl.BlockSpec((1, D), lambda i: (0, 0))
_bs_degp = pl.BlockSpec((2, B, D), lambda i: (0, i, 0))
_bs_p = pl.BlockSpec((2, B, D), lambda i: (0, i, 0))

_mm_first = pl.pallas_call(
    _mm_first_body, grid=_grid,
    in_specs=[_bs_x, _bs_w, _bs_degp], out_specs=_bs_x,
    out_shape=jax.ShapeDtypeStruct((NP, D), jnp.float32))

_mm_mid = pl.pallas_call(
    _mm_mid_body, grid=_grid,
    in_specs=[_bs_p, _bs_degp, _bs_b, _bs_w], out_specs=_bs_x,
    out_shape=jax.ShapeDtypeStruct((NP, D), jnp.float32))

_finish = pl.pallas_call(
    _finish_body, grid=_grid,
    in_specs=[_bs_p, _bs_degp, _bs_b], out_specs=_bs_x,
    out_shape=jax.ShapeDtypeStruct((NP, D), jnp.float32))


def kernel(x, edge_index, edge_weight, W1, b1, W2, b2, W3, b3, W4, b4):
    loop = jnp.arange(N, dtype=edge_index.dtype)
    row = jnp.concatenate([edge_index[0], loop])
    col = jnp.concatenate([edge_index[1], loop])
    ew = jnp.concatenate([edge_weight, jnp.ones((N,), jnp.float32)])
    pad = EPAD - (E + N)
    rowc = jnp.pad(row, (0, pad)).reshape(NCHUNK, K)
    colc = jnp.pad(col, (0, pad)).reshape(NCHUNK, K)
    ewbc = lax.bitcast_convert_type(
        jnp.pad(ew, (0, pad)), jnp.int32).reshape(NCHUNK, K)
    pkc = jnp.stack([rowc, colc, ewbc], axis=1)
    n0 = NS * Q0
    pk_a = jnp.pad(pkc[:n0].reshape(NS, Q0, 3, K),
                   ((0, 0), (0, NITM - Q0), (0, 0), (0, 0)))
    pk_b = jnp.pad(pkc[n0:].reshape(NS, Q1, 3, K),
                   ((0, 0), (0, NITM - Q1), (0, 0), (0, 0)))
    pk3 = jnp.stack([pk_a, pk_b], axis=1).reshape(NW, NITM, 3, K)
    pk3b = pkc.reshape(NW, QB, 3, K)
    xp = jnp.pad(x, ((0, NP - N), (0, 0)))

    degp = _deg_call(pk3b)

    b1r, b2r, b3r, b4r = (b.reshape(1, D) for b in (b1, b2, b3, b4))
    z = _mm_first(xp, W1, degp)
    p = _agg_call(z, pk3)
    z = _mm_mid(p, degp, b1r, W2)
    p = _agg_call(z, pk3)
    z = _mm_mid(p, degp, b2r, W3)
    p = _agg_call(z, pk3)
    z = _mm_mid(p, degp, b3r, W4)
    p = _agg_call(z, pk3)
    return _finish(p, degp, b4r)[:N]
